# baseline (speedup 1.0000x reference)
import functools

import jax
import jax.numpy as jnp
from jax import lax
from jax.experimental import pallas as pl
from jax.experimental.pallas import tpu as pltpu
from jax.experimental.pallas import tpu_sc as plsc

NC = 2
NS = 16
NW = NC * NS

SUB = 128
NSUB = 4
CH = SUB * NSUB


@functools.partial(jax.jit, static_argnums=(2, 3))
def _sc_gather(idx2d, table, N, D):
    per_w = N // NW
    n_chunks = per_w // CH
    mesh = plsc.VectorSubcoreMesh(core_axis_name="c", subcore_axis_name="s")

    @functools.partial(
        pl.kernel,
        mesh=mesh,
        compiler_params=pltpu.CompilerParams(use_tc_tiling_on_sc=False),
        out_type=jax.ShapeDtypeStruct((N, D), jnp.float32),
        scratch_types=[
            pltpu.VMEM((NSUB, SUB), jnp.int32),
            pltpu.VMEM((CH, D), jnp.float32),
            pltpu.SemaphoreType.DMA,
        ],
    )
    def k(idx_hbm, table_hbm, out_hbm, idx_v, rows_v, gsem):
        wid = lax.axis_index("s") * NC + lax.axis_index("c")
        wrow = wid * (per_w // SUB)

        def body(c, carry):
            r0 = wrow + c * NSUB
            pltpu.sync_copy(idx_hbm.at[pl.ds(r0, NSUB)], idx_v)
            handles = [
                pltpu.async_copy(
                    table_hbm.at[idx_v.at[j]],
                    rows_v.at[pl.ds(j * SUB, SUB)],
                    gsem,
                )
                for j in range(NSUB)
            ]
            for h in handles:
                h.wait()
            base = wid * per_w + c * CH
            pltpu.sync_copy(rows_v, out_hbm.at[pl.ds(base, CH)])
            return carry

        lax.fori_loop(0, n_chunks, body, 0)

    return k(idx2d, table)


def kernel(x, table):
    B0, B1 = x.shape
    _, D = table.shape
    N = B0 * B1
    idx2d = x.astype(jnp.int32).reshape(N // SUB, SUB)
    out = _sc_gather(idx2d, table, N, D)
    return out.reshape(B0, B1, D)


# speedup vs baseline: 1.0449x; 1.0449x over previous
import functools

import jax
import jax.numpy as jnp
from jax import lax
from jax.experimental import pallas as pl
from jax.experimental.pallas import tpu as pltpu
from jax.experimental.pallas import tpu_sc as plsc

NC = 2
NS = 16
NW = NC * NS

SUB = 128
NSUB = 4
CH = SUB * NSUB


@functools.partial(jax.jit, static_argnums=(2, 3))
def _sc_gather(idx2d, table, N, D):
    per_w = N // NW
    rows_w = per_w // SUB
    n_chunks = per_w // CH
    mesh = plsc.VectorSubcoreMesh(core_axis_name="c", subcore_axis_name="s")

    @functools.partial(
        pl.kernel,
        mesh=mesh,
        compiler_params=pltpu.CompilerParams(use_tc_tiling_on_sc=False),
        out_type=jax.ShapeDtypeStruct((N, D), jnp.float32),
        scratch_types=[
            pltpu.VMEM((rows_w, SUB), jnp.int32),
            pltpu.VMEM((2, CH, D), jnp.float32),
            pltpu.SemaphoreType.DMA,
            pltpu.SemaphoreType.DMA,
        ],
    )
    def k(idx_hbm, table_hbm, out_hbm, idx_v, rows_v, gsem, ssem):
        wid = lax.axis_index("s") * NC + lax.axis_index("c")
        wbase = wid * per_w

        pltpu.sync_copy(idx_hbm.at[pl.ds(wid * rows_w, rows_w)], idx_v)

        def fire_gathers(c, b):
            return [
                pltpu.async_copy(
                    table_hbm.at[idx_v.at[c * NSUB + j]],
                    rows_v.at[b].at[pl.ds(j * SUB, SUB)],
                    gsem,
                )
                for j in range(NSUB)
            ]

        def store(c, b):
            return pltpu.async_copy(
                rows_v.at[b], out_hbm.at[pl.ds(wbase + c * CH, CH)], ssem
            )

        for h in fire_gathers(0, 0):
            h.wait()
        store(0, 0)
        fire_gathers(1, 1)

        def pair_body(p, carry):
            for b, i in ((1, 2 * p + 1), (0, 2 * p + 2)):
                for j in range(NSUB):
                    pltpu.make_async_copy(
                        table_hbm.at[idx_v.at[i * NSUB + j]],
                        rows_v.at[b].at[pl.ds(j * SUB, SUB)],
                        gsem,
                    ).wait()
                pltpu.make_async_copy(
                    rows_v.at[1 - b],
                    out_hbm.at[pl.ds(wbase + (i - 1) * CH, CH)],
                    ssem,
                ).wait()
                fire_gathers(i + 1, 1 - b)
                store(i, b)
            return carry

        lax.fori_loop(0, (n_chunks - 2) // 2, pair_body, 0)

        last = n_chunks - 1
        lb = last % 2
        for j in range(NSUB):
            pltpu.make_async_copy(
                table_hbm.at[idx_v.at[last * NSUB + j]],
                rows_v.at[lb].at[pl.ds(j * SUB, SUB)],
                gsem,
            ).wait()
        pltpu.make_async_copy(
            rows_v.at[1 - lb],
            out_hbm.at[pl.ds(wbase + (last - 1) * CH, CH)],
            ssem,
        ).wait()
        store(last, lb).wait()

    return k(idx2d, table)


def kernel(x, table):
    B0, B1 = x.shape
    _, D = table.shape
    N = B0 * B1
    idx2d = x.astype(jnp.int32).reshape(N // SUB, SUB)
    out = _sc_gather(idx2d, table, N, D)
    return out.reshape(B0, B1, D)
